# baseline (speedup 1.0000x reference)
import functools

import jax
import jax.numpy as jnp
from jax import lax
from jax.experimental import pallas as pl
from jax.experimental.pallas import tpu as pltpu
from jax.experimental.pallas import tpu_sc as plsc

N = 10000
E = 320000
NPAD = 10240
D = 64
NC, NS = 2, 16
NW = NC * NS
GROUP = 125
G = 80
NBUF = 8
PFD = 4
RPS = NPAD // NS
IN_CH_ = 128
EPS = 1e-5

_mesh = plsc.VectorSubcoreMesh(
    core_axis_name="c", subcore_axis_name="s", num_cores=NC, num_subcores=NS)
_sc_params = pltpu.CompilerParams(use_tc_tiling_on_sc=False)


@functools.partial(
    pl.kernel,
    out_type=jax.ShapeDtypeStruct((NC, NPAD, 4, 16), jnp.float32),
    mesh=_mesh,
    compiler_params=_sc_params,
    scratch_types=[
        pltpu.VMEM((G, GROUP), jnp.int32),
        pltpu.VMEM((GROUP, 16), jnp.float32),
        pltpu.VMEM((128, 16), jnp.float32),
        pltpu.VMEM_SHARED((NPAD, 16), jnp.float32),
        pltpu.SemaphoreType.DMA,
    ],
)
def _deg_kernel(edge_hbm, ones_hbm, zeros_hbm, deg_hbm, didx_v, ones_v, zz_v,
                acc_sh, sem):
    cid = lax.axis_index("c")
    sid = lax.axis_index("s")
    wid = sid * NC + cid
    pltpu.sync_copy(edge_hbm.at[1, wid], didx_v)
    pltpu.sync_copy(ones_hbm, ones_v)
    pltpu.sync_copy(zeros_hbm, zz_v)

    @pl.loop(0, RPS // 128)
    def _zero(k):
        pltpu.sync_copy(zz_v, acc_sh.at[pl.ds(sid * RPS + k * 128, 128)])

    plsc.subcore_barrier()

    @pl.loop(0, G)
    def _scat(g):
        pltpu.sync_copy(ones_v, acc_sh.at[didx_v.at[g]], add=True)

    plsc.subcore_barrier()

    @pl.loop(0, RPS // 128)
    def _out(k):
        row = sid * RPS + k * 128
        for j in range(4):
            pltpu.sync_copy(acc_sh.at[pl.ds(row, 128)],
                            deg_hbm.at[cid, pl.ds(row, 128), j])


@functools.partial(
    pl.kernel,
    out_type=jax.ShapeDtypeStruct((NC, NPAD, D), jnp.float32),
    mesh=_mesh,
    compiler_params=_sc_params,
    scratch_types=[
        pltpu.VMEM((G, GROUP), jnp.int32),
        pltpu.VMEM((G, GROUP), jnp.int32),
        pltpu.VMEM((NBUF, GROUP, D), jnp.float32),
        pltpu.VMEM((64, D), jnp.float32),
        pltpu.VMEM_SHARED((NPAD, D), jnp.float32),
        [pltpu.SemaphoreType.DMA] * NBUF,
        [pltpu.SemaphoreType.DMA] * NBUF,
    ],
)
def _conv_kernel(hs_hbm, edge_hbm, z_hbm, out_hbm,
                 sidx_v, didx_v, rows_v, zz_v, acc_sh, gsems, ssems):
    cid = lax.axis_index("c")
    sid = lax.axis_index("s")
    wid = sid * NC + cid
    pltpu.sync_copy(edge_hbm.at[0, wid], sidx_v)
    pltpu.sync_copy(edge_hbm.at[1, wid], didx_v)
    pltpu.sync_copy(z_hbm, zz_v)

    @pl.loop(0, RPS // 64)
    def _zero(k):
        pltpu.sync_copy(zz_v, acc_sh.at[pl.ds(sid * RPS + k * 64, 64)])

    plsc.subcore_barrier()

    for b in range(PFD):
        pltpu.async_copy(hs_hbm.at[sidx_v.at[b]], rows_v.at[b], gsems[b])

    @pl.loop(0, G, step=NBUF)
    def _edges(g):
        for b in range(NBUF):
            k = g + b
            bn = (b + PFD) % NBUF

            @pl.when(k >= PFD)
            def _():
                pltpu.make_async_copy(
                    rows_v.at[bn], acc_sh.at[didx_v.at[0]], ssems[bn]).wait()

            @pl.when(k + PFD < G)
            def _():
                pltpu.async_copy(
                    hs_hbm.at[sidx_v.at[k + PFD]], rows_v.at[bn], gsems[bn])

            pltpu.make_async_copy(
                hs_hbm.at[sidx_v.at[b]], rows_v.at[b], gsems[b]).wait()
            pltpu.async_copy(
                rows_v.at[b], acc_sh.at[didx_v.at[k]], ssems[b], add=True)

    for b in range(NBUF - PFD, NBUF):
        pltpu.make_async_copy(
            rows_v.at[b], acc_sh.at[didx_v.at[0]], ssems[b]).wait()

    plsc.subcore_barrier()

    @pl.loop(0, RPS // 128)
    def _out(k):
        row = sid * RPS + k * 128
        pltpu.sync_copy(acc_sh.at[pl.ds(row, 128)],
                        out_hbm.at[cid, pl.ds(row, 128)])


NP = N // 2
NPP = NPAD // 2


def _bn_relu_packed(t, g2, be2):
    mu1 = jnp.mean(t, axis=0, keepdims=True)
    m21 = jnp.mean(t * t, axis=0, keepdims=True)
    mu = (mu1[:, :D] + mu1[:, D:]) * 0.5
    m2 = (m21[:, :D] + m21[:, D:]) * 0.5
    mu = jnp.concatenate([mu, mu], axis=1)
    m2 = jnp.concatenate([m2, m2], axis=1)
    s = lax.rsqrt(m2 - mu * mu + EPS)
    a = s * g2
    return jnp.maximum(t * a + (be2 - mu * a), 0.0)


def _tc_m(x_ref, w_ref, h_ref):
    h_ref[...] = jnp.dot(x_ref[...], w_ref[...],
                         preferred_element_type=jnp.float32)


def _tc_a(h_ref, deg_ref, hs_ref, dinv_ref):
    degsum = deg_ref[0, :NP] + deg_ref[1, :NP] + 1.0
    dinv = lax.rsqrt(degsum)
    dinv_ref[...] = dinv
    hs_ref[...] = h_ref[...] * dinv


def _tc_b(s_ref, hs_ref, dinv_ref, g_ref, be_ref, w_ref, out_ref):
    dinv = dinv_ref[...]
    t = (s_ref[0, :NP] + s_ref[1, :NP] + hs_ref[...]) * dinv
    t = _bn_relu_packed(t, g_ref[...], be_ref[...])
    out_ref[...] = jnp.dot(t, w_ref[...],
                           preferred_element_type=jnp.float32) * dinv


def _tc_c(s_ref, hs_ref, dinv_ref, g_ref, be_ref,
          pw1_ref, pg_ref, pbe_ref, pw2_ref, pb2_ref, ones_ref, out_ref):
    dinv = dinv_ref[...]
    t = (s_ref[0, :NP] + s_ref[1, :NP] + hs_ref[...]) * dinv
    t = _bn_relu_packed(t, g_ref[...], be_ref[...])
    p = jnp.dot(t, pw1_ref[...], preferred_element_type=jnp.float32)
    p = _bn_relu_packed(p, pg_ref[...], pbe_ref[...])
    q = jnp.dot(p, pw2_ref[...], preferred_element_type=jnp.float32)
    q = q + pb2_ref[...]
    q2 = q * q
    ss = jnp.dot(q2, ones_ref[...], preferred_element_type=jnp.float32)
    out_ref[...] = q * lax.rsqrt(jnp.maximum(ss, 1e-24))


_f32 = jnp.float32
_tc_m_call = pl.pallas_call(
    _tc_m, out_shape=jax.ShapeDtypeStruct((NP, 2 * D), _f32))
_tc_a_call = pl.pallas_call(
    _tc_a, out_shape=(jax.ShapeDtypeStruct((NP, 2 * D), _f32),
                      jax.ShapeDtypeStruct((NP, 2 * D), _f32)))
_tc_b_call = pl.pallas_call(
    _tc_b, out_shape=jax.ShapeDtypeStruct((NP, 2 * D), _f32))
_tc_c_call = pl.pallas_call(
    _tc_c, out_shape=jax.ShapeDtypeStruct((NP, 2 * D), _f32))


def _blockdiag(w):
    z = jnp.zeros_like(w)
    return jnp.concatenate(
        [jnp.concatenate([w, z], axis=1), jnp.concatenate([z, w], axis=1)],
        axis=0)


def kernel(x, edge_index, W1, b1, g1, be1, W2, b2, g2, be2,
           Pw1, Pb1, Pg, Pbe, Pw2, Pb2):
    edge4 = edge_index.astype(jnp.int32).reshape(2, NW, G, GROUP)
    ones16 = jnp.ones((GROUP, 16), _f32)
    zeros16 = jnp.zeros((128, 16), _f32)
    zeros64 = jnp.zeros((64, D), _f32)
    dup = lambda v: jnp.concatenate([v, v]).reshape(1, 2 * D)
    g1r, be1r = dup(g1), dup(be1)
    g2r, be2r = dup(g2), dup(be2)
    pgr, pber, pb2r = dup(Pg), dup(Pbe), dup(Pb2)
    w1d = _blockdiag(W1)
    w2d = _blockdiag(W2)
    pw1d = _blockdiag(Pw1)
    pw2d = _blockdiag(Pw2)
    xp = x.reshape(NP, 2 * IN_CH_)

    deg = _deg_kernel(edge4, ones16, zeros16)
    h1 = _tc_m_call(xp, w1d)
    hs1, dinv_p = _tc_a_call(h1, deg.reshape(NC, NPP, 2 * D))
    s1 = _conv_kernel(hs1.reshape(N, D), edge4, zeros64)
    hs2 = _tc_b_call(s1.reshape(NC, NPP, 2 * D), hs1, dinv_p, g1r, be1r, w2d)
    s2 = _conv_kernel(hs2.reshape(N, D), edge4, zeros64)
    onesd = _blockdiag(jnp.ones((D, D), _f32))
    out = _tc_c_call(s2.reshape(NC, NPP, 2 * D), hs2, dinv_p,
                     g2r, be2r, pw1d, pgr, pber, pw2d, pb2r, onesd)
    return out.reshape(N, D)


# speedup vs baseline: 1.0271x; 1.0271x over previous
import functools

import jax
import jax.numpy as jnp
from jax import lax
from jax.experimental import pallas as pl
from jax.experimental.pallas import tpu as pltpu
from jax.experimental.pallas import tpu_sc as plsc

N = 10000
E = 320000
NPAD = 10240
D = 64
IN_CH_ = 128
NC, NS = 2, 16
NW = NC * NS
GROUP = 125
G = 80
NBUF = 8
RPS = NPAD // NS
EPS = 1e-5

_mesh = plsc.VectorSubcoreMesh(
    core_axis_name="c", subcore_axis_name="s", num_cores=NC, num_subcores=NS)
_sc_params = pltpu.CompilerParams(use_tc_tiling_on_sc=False)


@functools.partial(
    pl.kernel,
    out_type=jax.ShapeDtypeStruct((NC, NPAD, 4, 16), jnp.float32),
    mesh=_mesh,
    compiler_params=_sc_params,
    scratch_types=[
        pltpu.VMEM((G, GROUP), jnp.int32),
        pltpu.VMEM((GROUP, 16), jnp.float32),
        pltpu.VMEM((128, 16), jnp.float32),
        pltpu.VMEM_SHARED((NPAD, 16), jnp.float32),
    ],
)
def _deg_kernel(edge_hbm, ones_hbm, zeros_hbm, deg_hbm, didx_v, ones_v, zz_v,
                acc_sh):
    cid = lax.axis_index("c")
    sid = lax.axis_index("s")
    wid = sid * NC + cid
    pltpu.sync_copy(edge_hbm.at[1, wid], didx_v)
    pltpu.sync_copy(ones_hbm, ones_v)
    pltpu.sync_copy(zeros_hbm, zz_v)

    @pl.loop(0, RPS // 128)
    def _zero(k):
        pltpu.sync_copy(zz_v, acc_sh.at[pl.ds(sid * RPS + k * 128, 128)])

    plsc.subcore_barrier()

    @pl.loop(0, G)
    def _scat(g):
        pltpu.sync_copy(ones_v, acc_sh.at[didx_v.at[g]], add=True)

    plsc.subcore_barrier()

    @pl.loop(0, RPS // 128)
    def _out(k):
        row = sid * RPS + k * 128
        for j in range(4):
            pltpu.sync_copy(acc_sh.at[pl.ds(row, 128)],
                            deg_hbm.at[cid, pl.ds(row, 128), j])


@functools.partial(
    pl.kernel,
    out_type=jax.ShapeDtypeStruct((NC, NPAD, D), jnp.float32),
    mesh=_mesh,
    compiler_params=_sc_params,
    scratch_types=[
        pltpu.VMEM((G, GROUP), jnp.int32),
        pltpu.VMEM((G, GROUP), jnp.int32),
        pltpu.VMEM((NBUF, GROUP, D), jnp.float32),
        pltpu.VMEM((64, D), jnp.float32),
        pltpu.VMEM_SHARED((NPAD, D), jnp.float32),
        [pltpu.SemaphoreType.DMA] * NBUF,
    ],
)
def _conv_kernel(hs_hbm, edge_hbm, z_hbm, out_hbm,
                 sidx_v, didx_v, rows_v, zz_v, acc_sh, sems):
    cid = lax.axis_index("c")
    sid = lax.axis_index("s")
    wid = sid * NC + cid
    pltpu.sync_copy(edge_hbm.at[0, wid], sidx_v)
    pltpu.sync_copy(edge_hbm.at[1, wid], didx_v)
    pltpu.sync_copy(z_hbm, zz_v)

    @pl.loop(0, RPS // 64)
    def _zero(k):
        pltpu.sync_copy(zz_v, acc_sh.at[pl.ds(sid * RPS + k * 64, 64)])

    plsc.subcore_barrier()

    for b in range(NBUF):
        pltpu.async_copy(hs_hbm.at[sidx_v.at[b]], rows_v.at[b], sems[b])

    @pl.loop(0, G, step=NBUF)
    def _edges(g):
        for b in range(NBUF):
            pltpu.make_async_copy(
                hs_hbm.at[sidx_v.at[b]], rows_v.at[b], sems[b]).wait()
            pltpu.sync_copy(rows_v.at[b], acc_sh.at[didx_v.at[g + b]], add=True)

            @pl.when(g + b + NBUF < G)
            def _():
                pltpu.async_copy(
                    hs_hbm.at[sidx_v.at[g + b + NBUF]], rows_v.at[b], sems[b])

    plsc.subcore_barrier()

    @pl.loop(0, RPS // 128)
    def _out(k):
        row = sid * RPS + k * 128
        pltpu.sync_copy(acc_sh.at[pl.ds(row, 128)],
                        out_hbm.at[cid, pl.ds(row, 128)])


NP = N // 2
NPP = NPAD // 2


def _bn_relu_packed(t, g2, be2):
    mu1 = jnp.mean(t, axis=0, keepdims=True)
    m21 = jnp.mean(t * t, axis=0, keepdims=True)
    mu = (mu1[:, :D] + mu1[:, D:]) * 0.5
    m2 = (m21[:, :D] + m21[:, D:]) * 0.5
    mu = jnp.concatenate([mu, mu], axis=1)
    m2 = jnp.concatenate([m2, m2], axis=1)
    s = lax.rsqrt(m2 - mu * mu + EPS)
    a = s * g2
    return jnp.maximum(t * a + (be2 - mu * a), 0.0)


def _tc_m(x_ref, w_ref, h_ref):
    h_ref[...] = jnp.dot(x_ref[...], w_ref[...],
                         preferred_element_type=jnp.float32)


def _tc_a(h_ref, deg_ref, hs_ref, dinv_ref):
    degsum = deg_ref[0, :NP] + deg_ref[1, :NP] + 1.0
    dinv = lax.rsqrt(degsum)
    dinv_ref[...] = dinv
    hs_ref[...] = h_ref[...] * dinv


def _tc_b(s_ref, hs_ref, dinv_ref, g_ref, be_ref, w_ref, out_ref):
    dinv = dinv_ref[...]
    t = (s_ref[0, :NP] + s_ref[1, :NP] + hs_ref[...]) * dinv
    t = _bn_relu_packed(t, g_ref[...], be_ref[...])
    out_ref[...] = jnp.dot(t, w_ref[...],
                           preferred_element_type=jnp.float32) * dinv


def _tc_c(s_ref, hs_ref, dinv_ref, g_ref, be_ref,
          pw1_ref, pg_ref, pbe_ref, pw2_ref, pb2_ref, ones_ref, out_ref):
    dinv = dinv_ref[...]
    t = (s_ref[0, :NP] + s_ref[1, :NP] + hs_ref[...]) * dinv
    t = _bn_relu_packed(t, g_ref[...], be_ref[...])
    p = jnp.dot(t, pw1_ref[...], preferred_element_type=jnp.float32)
    p = _bn_relu_packed(p, pg_ref[...], pbe_ref[...])
    q = jnp.dot(p, pw2_ref[...], preferred_element_type=jnp.float32)
    q = q + pb2_ref[...]
    q2 = q * q
    ss = jnp.dot(q2, ones_ref[...], preferred_element_type=jnp.float32)
    out_ref[...] = q * lax.rsqrt(jnp.maximum(ss, 1e-24))


_f32 = jnp.float32
_tc_m_call = pl.pallas_call(
    _tc_m, out_shape=jax.ShapeDtypeStruct((NP, 2 * D), _f32))
_tc_a_call = pl.pallas_call(
    _tc_a, out_shape=(jax.ShapeDtypeStruct((NP, 2 * D), _f32),
                      jax.ShapeDtypeStruct((NP, 2 * D), _f32)))
_tc_b_call = pl.pallas_call(
    _tc_b, out_shape=jax.ShapeDtypeStruct((NP, 2 * D), _f32))
_tc_c_call = pl.pallas_call(
    _tc_c, out_shape=jax.ShapeDtypeStruct((NP, 2 * D), _f32))


def _blockdiag(w):
    z = jnp.zeros_like(w)
    return jnp.concatenate(
        [jnp.concatenate([w, z], axis=1), jnp.concatenate([z, w], axis=1)],
        axis=0)


def kernel(x, edge_index, W1, b1, g1, be1, W2, b2, g2, be2,
           Pw1, Pb1, Pg, Pbe, Pw2, Pb2):
    edge4 = edge_index.astype(jnp.int32).reshape(2, NW, G, GROUP)
    ones16 = jnp.ones((GROUP, 16), _f32)
    zeros16 = jnp.zeros((128, 16), _f32)
    zeros64 = jnp.zeros((64, D), _f32)
    dup = lambda v: jnp.concatenate([v, v]).reshape(1, 2 * D)
    g1r, be1r = dup(g1), dup(be1)
    g2r, be2r = dup(g2), dup(be2)
    pgr, pber, pb2r = dup(Pg), dup(Pbe), dup(Pb2)
    w1d = _blockdiag(W1)
    w2d = _blockdiag(W2)
    pw1d = _blockdiag(Pw1)
    pw2d = _blockdiag(Pw2)
    onesd = _blockdiag(jnp.ones((D, D), _f32))
    xp = x.reshape(NP, 2 * IN_CH_)

    deg = _deg_kernel(edge4, ones16, zeros16)
    h1 = _tc_m_call(xp, w1d)
    hs1, dinv_p = _tc_a_call(h1, deg.reshape(NC, NPP, 2 * D))
    s1 = _conv_kernel(hs1.reshape(N, D), edge4, zeros64)
    hs2 = _tc_b_call(s1.reshape(NC, NPP, 2 * D), hs1, dinv_p, g1r, be1r, w2d)
    s2 = _conv_kernel(hs2.reshape(N, D), edge4, zeros64)
    out = _tc_c_call(s2.reshape(NC, NPP, 2 * D), hs2, dinv_p,
                     g2r, be2r, pw1d, pgr, pber, pw2d, pb2r, onesd)
    return out.reshape(N, D)
